# baseline (speedup 1.0000x reference)
import functools

import jax
import jax.numpy as jnp
from jax.experimental import pallas as pl
from jax.experimental.pallas import tpu as pltpu

_D = 64
_M = 512
_K = 8
_TAU = 0.1
_SUB = 512
_NSUB = 16
_BLK = _SUB * _NSUB


def _body(q_ref, mem_ref, mem_aug_ref, out_ref, smax_ref):
    mem = mem_ref[...]
    mem_aug = mem_aug_ref[...]
    mn = mem * jax.lax.rsqrt(jnp.maximum(jnp.sum(mem * mem, axis=1, keepdims=True), 1e-24))

    for h in range(_NSUB):
        rows = pl.ds(h * _SUB, _SUB)
        q = q_ref[0, rows, :]
        qn = q * jax.lax.rsqrt(jnp.maximum(jnp.sum(q * q, axis=1, keepdims=True), 1e-24))

        sim = jnp.dot(qn, mn.T, preferred_element_type=jnp.float32)

        neg = jnp.float32(-jnp.inf)
        smax = jnp.max(sim, axis=1, keepdims=True)
        thr = smax
        for _ in range(_K - 1):
            thr = jnp.max(jnp.where(sim < thr, sim, neg), axis=1, keepdims=True)

        smax_s = smax * (1.0 / _TAU)
        e = jnp.where(sim >= thr, jnp.exp(sim * (1.0 / _TAU) - smax_s), 0.0)

        r = jnp.dot(e, mem_aug, preferred_element_type=jnp.float32)
        out_ref[0, rows, :] = r[:, :_D] * (1.0 / r[:, _D:])
        smax_ref[0, rows, :] = smax_s


@functools.partial(jax.jit, static_argnames=())
def kernel(queries, memory):
    b, n, d = queries.shape
    mem_aug = jnp.concatenate(
        [memory, jnp.ones((memory.shape[0], 1), jnp.float32)], axis=1)
    grid = (b,)
    out, smax = pl.pallas_call(
        _body,
        grid=grid,
        in_specs=[
            pl.BlockSpec((1, _BLK, d), lambda i: (i, 0, 0)),
            pl.BlockSpec((_M, d), lambda i: (0, 0)),
            pl.BlockSpec((_M, d + 1), lambda i: (0, 0)),
        ],
        out_specs=[
            pl.BlockSpec((1, _BLK, d), lambda i: (i, 0, 0)),
            pl.BlockSpec((1, _BLK, 1), lambda i: (i, 0, 0)),
        ],
        out_shape=[
            jax.ShapeDtypeStruct((b, n, d), jnp.float32),
            jax.ShapeDtypeStruct((b, n, 1), jnp.float32),
        ],
        compiler_params=pltpu.CompilerParams(
            dimension_semantics=("parallel",),
        ),
    )(queries, memory, mem_aug)
    return out, smax


# speedup vs baseline: 1.1178x; 1.1178x over previous
import functools

import jax
import jax.numpy as jnp
from jax.experimental import pallas as pl
from jax.experimental.pallas import tpu as pltpu

_D = 64
_M = 512
_K = 8
_TAU = 0.1
_SUB = 512
_NSUB = 4
_BLK = _SUB * _NSUB


def _body(q_ref, mem_ref, mem_aug_ref, out_ref, smax_ref):
    mem = mem_ref[...]
    mem_aug = mem_aug_ref[...]
    mn = mem * jax.lax.rsqrt(jnp.maximum(jnp.sum(mem * mem, axis=1, keepdims=True), 1e-24))

    for h in range(_NSUB):
        rows = pl.ds(h * _SUB, _SUB)
        q = q_ref[rows, :]
        qn = q * jax.lax.rsqrt(jnp.maximum(jnp.sum(q * q, axis=1, keepdims=True), 1e-24))

        sim = jnp.dot(qn, mn.T, preferred_element_type=jnp.float32)

        neg = jnp.float32(-jnp.inf)
        smax = jnp.max(sim, axis=1, keepdims=True)
        thr = smax
        for _ in range(_K - 1):
            thr = jnp.max(jnp.where(sim < thr, sim, neg), axis=1, keepdims=True)

        smax_s = smax * (1.0 / _TAU)
        e = jnp.where(sim >= thr, jnp.exp(sim * (1.0 / _TAU) - smax_s), 0.0)

        r = jnp.dot(e, mem_aug, preferred_element_type=jnp.float32)
        out_ref[rows, :] = r[:, :_D] * (1.0 / r[:, _D:])
        smax_ref[rows, :] = smax_s


@functools.partial(jax.jit, static_argnames=())
def kernel(queries, memory):
    b, n, d = queries.shape
    tokens = b * n
    q2 = queries.reshape(tokens, d)
    mem_aug = jnp.concatenate(
        [memory, jnp.ones((memory.shape[0], 1), jnp.float32)], axis=1)
    grid = (tokens // _BLK,)
    out, smax = pl.pallas_call(
        _body,
        grid=grid,
        in_specs=[
            pl.BlockSpec((_BLK, d), lambda i: (i, 0)),
            pl.BlockSpec((_M, d), lambda i: (0, 0)),
            pl.BlockSpec((_M, d + 1), lambda i: (0, 0)),
        ],
        out_specs=[
            pl.BlockSpec((_BLK, d), lambda i: (i, 0)),
            pl.BlockSpec((_BLK, 1), lambda i: (i, 0)),
        ],
        out_shape=[
            jax.ShapeDtypeStruct((tokens, d), jnp.float32),
            jax.ShapeDtypeStruct((tokens, 1), jnp.float32),
        ],
        compiler_params=pltpu.CompilerParams(
            dimension_semantics=("parallel",),
        ),
    )(q2, memory, mem_aug)
    return out.reshape(b, n, d), smax.reshape(b, n, 1)
